# Initial kernel scaffold; baseline (speedup 1.0000x reference)
import jax
import jax.numpy as jnp
from jax.experimental import pallas as pl


def kernel(x, edge_index, params):
    raise NotImplementedError("write your pallas kernel here")


# speedup vs baseline: 6.5636x; 6.5636x over previous
import functools

import jax
import jax.numpy as jnp
from jax import lax
from jax.experimental import pallas as pl
from jax.experimental.pallas import tpu as pltpu
from jax.experimental.pallas import tpu_sc as plsc

B = 8
NPG = 1250
N = B * NPG
EPG = 20000
K = 4
PAD = 1280
FLAT = NPG * PAD
PASSES = 5
HROWS = NPG // PASSES
HFLAT = HROWS * PAD
TPH = HFLAT // 16
EPT = 1248
GROUPS = 80

F32 = jnp.float32
BF16 = jnp.bfloat16


def _counts_body(srch, dsth, zeros_hbm, out,
                 srcb, dstb, idxb, valsb, zbuf, obuf, acc):
    c = lax.axis_index("c")
    s = lax.axis_index("s")

    pltpu.sync_copy(zeros_hbm, zbuf)
    one16 = jnp.ones((16,), F32)
    ow_tail = jnp.zeros((16,), F32) + jnp.where(s == 15, 1.0, 0.0)

    def graph_body(gi, carry):
        g = c * 4 + gi
        base_node = g * NPG
        ebase = g * EPG + s * EPT

        pltpu.sync_copy(srch.at[pl.ds(ebase, 1280)], srcb)
        pltpu.sync_copy(dsth.at[pl.ds(ebase, 1280)], dstb)

        for j in range(GROUPS):
            sv = srcb[pl.ds(j * 16, 16)] - base_node
            dv = dstb[pl.ds(j * 16, 16)] - base_node
            ow = one16 if j < 78 else ow_tail
            for h in range(PASSES):
                r = sv - h * HROWS
                ok = (r >= 0) & (r < HROWS)
                rc = jnp.clip(r, 0, HROWS - 1)
                idxb[10 * h + j // 8, pl.ds((j % 8) * 16, 16)] = rc * PAD + dv
                valsb[10 * h + j // 8, pl.ds((j % 8) * 16, 16)] = (
                    jnp.where(ok, ow, 0.0))

        for h in range(PASSES):
            pltpu.sync_copy(zbuf,
                            acc.at[pl.ds(pl.multiple_of(s * TPH, 8), TPH)])
            plsc.subcore_barrier()
            for r in range(10):
                pltpu.sync_copy(valsb.at[10 * h + r],
                                acc.at[idxb.at[10 * h + r]], add=True)
            plsc.subcore_barrier()
            pltpu.sync_copy(acc.at[pl.ds(pl.multiple_of(s * TPH, 8), TPH)],
                            obuf)
            pltpu.sync_copy(
                obuf,
                out.at[pl.ds(
                    pl.multiple_of(g * FLAT + h * HFLAT + s * TPH, 8), TPH)])
        return carry

    lax.fori_loop(0, 4, graph_body, 0)


@functools.cache
def _counts_call():
    mesh = plsc.VectorSubcoreMesh(core_axis_name="c", subcore_axis_name="s")
    return functools.partial(
        pl.kernel,
        mesh=mesh,
        out_type=jax.ShapeDtypeStruct((B * FLAT,), F32),
        scratch_types=[
            pltpu.VMEM((1280,), jnp.int32),
            pltpu.VMEM((1280,), jnp.int32),
            pltpu.VMEM((10 * PASSES, 128), jnp.int32),
            pltpu.VMEM((10 * PASSES, 128), F32),
            pltpu.VMEM((TPH,), F32),
            pltpu.VMEM((TPH,), F32),
            pltpu.VMEM_SHARED((HFLAT,), F32),
        ],
    )(_counts_body)


def _build_counts(edge_index):
    zeros = jnp.zeros((TPH,), F32)
    flat = _counts_call()(edge_index[0], edge_index[1], zeros)
    return flat.reshape(B, NPG, PAD)


def _d(a, b, dims):
    return lax.dot_general(a.astype(BF16), b.astype(BF16), (dims, ((), ())),
                           preferred_element_type=F32)


def _split(x):
    hi = x.astype(BF16)
    return hi, (x - hi.astype(F32)).astype(BF16)


def _d3(a, b, dims):
    ah, al = _split(a)
    bh, bl = _split(b)
    return _d(ah, bh, dims) + _d(ah, bl, dims) + _d(al, bh, dims)


def _dsplit(ah, al, b, dims):
    bh, bl = _split(b)
    return _d(ah, bh, dims) + _d(ah, bl, dims) + _d(al, bh, dims)


def _rsqrt(v):
    r = lax.rsqrt(v)
    r = r * (1.5 - 0.5 * v * r * r)
    return r * (1.5 - 0.5 * v * r * r)


def _recip(b):
    r = 1.0 / b
    r = r * (2.0 - b * r)
    return r * (2.0 - b * r)


def _bn(y, g, b):
    inv_n = 1.0 / y.shape[0]
    mu = jnp.sum(y, axis=0) * inv_n
    var = jnp.sum((y - mu) ** 2, axis=0) * inv_n
    return g * (y - mu) * _rsqrt(var + 1e-5) + b


def _agg0_body(a_ref, xa_ref, o_ref):
    a = a_ref[0]
    xh, xl = _split(xa_ref[0])
    r = _d(a, xh, ((0,), (0,))) + _d(a, xl, ((0,), (0,)))
    o_ref[0] = r[:NPG]


def _agg0(A, xa):
    return pl.pallas_call(
        _agg0_body,
        grid=(B,),
        in_specs=[pl.BlockSpec((1, NPG, PAD), lambda i: (i, 0, 0)),
                  pl.BlockSpec((1, NPG, 4), lambda i: (i, 0, 0))],
        out_specs=pl.BlockSpec((1, NPG, 4), lambda i: (i, 0, 0)),
        out_shape=jax.ShapeDtypeStruct((B, NPG, 4), F32),
    )(A, xa.reshape(B, NPG, 4))


def _aggmean_body(a_ref, hh_ref, hl_ref, deg_ref, oh_ref, ol_ref):
    a = a_ref[0]
    r = _d(a, hh_ref[0], ((0,), (0,))) + _d(a, hl_ref[0], ((0,), (0,)))
    mean = r[:NPG] * _recip(jnp.maximum(deg_ref[0], 1.0))
    hi, lo = _split(mean)
    oh_ref[0] = hi
    ol_ref[0] = lo


def _aggmean(A, h_hi, h_lo, deg):
    C = h_hi.shape[-1]
    outs = pl.pallas_call(
        _aggmean_body,
        grid=(B,),
        in_specs=[pl.BlockSpec((1, NPG, PAD), lambda i: (i, 0, 0)),
                  pl.BlockSpec((1, NPG, C), lambda i: (i, 0, 0)),
                  pl.BlockSpec((1, NPG, C), lambda i: (i, 0, 0)),
                  pl.BlockSpec((1, NPG, 1), lambda i: (i, 0, 0))],
        out_specs=[pl.BlockSpec((1, NPG, C), lambda i: (i, 0, 0)),
                   pl.BlockSpec((1, NPG, C), lambda i: (i, 0, 0))],
        out_shape=[jax.ShapeDtypeStruct((B, NPG, C), BF16),
                   jax.ShapeDtypeStruct((B, NPG, C), BF16)],
    )(A, h_hi.reshape(B, NPG, C), h_lo.reshape(B, NPG, C), deg)
    return outs[0].reshape(N, C), outs[1].reshape(N, C)


def _layer1_body(agg0_ref, x_ref, wl_ref, wr_ref, b_ref,
                 g_ref, bb_ref, oh_ref, ol_ref):
    mean = agg0_ref[:, :3] * _recip(jnp.maximum(agg0_ref[:, 3:4], 1.0))
    y = (_d(mean, wl_ref[...], ((1,), (0,)))
         + _d(x_ref[...], wr_ref[...], ((1,), (0,)))
         + b_ref[...])
    y = _bn(y, g_ref[...], bb_ref[...])
    hi, lo = _split(y)
    oh_ref[...] = hi
    ol_ref[...] = lo


def _layer1(agg0, x, wl, wr, b, g, bb):
    cout = wl.shape[1]
    nb = cout // 128
    outs = pl.pallas_call(
        _layer1_body,
        grid=(nb,),
        in_specs=[pl.BlockSpec((N, 4), lambda i: (0, 0)),
                  pl.BlockSpec((N, 3), lambda i: (0, 0)),
                  pl.BlockSpec((3, 128), lambda i: (0, i)),
                  pl.BlockSpec((3, 128), lambda i: (0, i)),
                  pl.BlockSpec((128,), lambda i: (i,)),
                  pl.BlockSpec((128,), lambda i: (i,)),
                  pl.BlockSpec((128,), lambda i: (i,))],
        out_specs=[pl.BlockSpec((N, 128), lambda i: (0, i)),
                   pl.BlockSpec((N, 128), lambda i: (0, i))],
        out_shape=[jax.ShapeDtypeStruct((N, cout), BF16),
                   jax.ShapeDtypeStruct((N, cout), BF16)],
    )(agg0, x, wl, wr, b, g, bb)
    return outs


def _layer_body(mh_ref, ml_ref, hh_ref, hl_ref, wl_ref, wr_ref, b_ref,
                g_ref, bb_ref, oh_ref, ol_ref):
    y = (_d(mh_ref[...], wl_ref[...], ((1,), (0,)))
         + _d(hh_ref[...], wr_ref[...], ((1,), (0,)))
         + b_ref[...])
    y = _bn(y, g_ref[...], bb_ref[...])
    hi, lo = _split(y)
    oh_ref[...] = hi
    ol_ref[...] = lo


def _layer(m_hi, m_lo, h_hi, h_lo, wl, wr, b, g, bb):
    cin = m_hi.shape[1]
    cout = wl.shape[1]
    nb = cout // 128
    outs = pl.pallas_call(
        _layer_body,
        grid=(nb,),
        in_specs=[pl.BlockSpec((N, cin), lambda i: (0, 0)),
                  pl.BlockSpec((N, cin), lambda i: (0, 0)),
                  pl.BlockSpec((N, cin), lambda i: (0, 0)),
                  pl.BlockSpec((N, cin), lambda i: (0, 0)),
                  pl.BlockSpec((cin, 128), lambda i: (0, i)),
                  pl.BlockSpec((cin, 128), lambda i: (0, i)),
                  pl.BlockSpec((128,), lambda i: (i,)),
                  pl.BlockSpec((128,), lambda i: (i,)),
                  pl.BlockSpec((128,), lambda i: (i,))],
        out_specs=[pl.BlockSpec((N, 128), lambda i: (0, i)),
                   pl.BlockSpec((N, 128), lambda i: (0, i))],
        out_shape=[jax.ShapeDtypeStruct((N, cout), BF16),
                   jax.ShapeDtypeStruct((N, cout), BF16)],
    )(m_hi, m_lo, h_hi, h_lo, wl, wr, b, g, bb)
    return outs


def _assign_body(mh_ref, ml_ref, hh_ref, hl_ref, wl_ref, wr_ref, b_ref,
                 g_ref, bb_ref, oh_ref, ol_ref):
    y = (_d(mh_ref[...], wl_ref[...], ((1,), (0,)))
         + _d(hh_ref[...], wr_ref[...], ((1,), (0,)))
         + b_ref[...])
    y = _bn(y, g_ref[...], bb_ref[...])
    m = jnp.max(y, axis=-1, keepdims=True)
    e = jnp.exp(y - m)
    sm = e * _recip(jnp.sum(e, axis=-1, keepdims=True))
    hi, lo = _split(sm)
    oh_ref[...] = hi
    ol_ref[...] = lo


def _assign(m_hi, m_lo, h_hi, h_lo, wl, wr, b, g, bb):
    outs = pl.pallas_call(
        _assign_body,
        out_shape=[jax.ShapeDtypeStruct((N, K), BF16),
                   jax.ShapeDtypeStruct((N, K), BF16)],
    )(m_hi, m_lo, h_hi, h_lo, wl, wr, b, g, bb)
    return outs


def _diffpool_body(a_ref, sh_ref, sl_ref, xh_ref, xl_ref,
                   px_ref, padj_ref, st_ref):
    adj = jnp.minimum(a_ref[0], 1.0)
    sh, sl = sh_ref[0], sl_ref[0]
    xh, xl = xh_ref[0], xl_ref[0]
    adjb = adj.astype(BF16)
    cd = ((0,), (0,))
    t1 = _d(sh, adjb, cd)
    padj = _d(t1[:, :NPG], sh, ((1,), (0,)))
    px = _d(sh, xh, cd)
    gmat = _d(sh, sh, cd) + _d(sh, sl, cd) + _d(sl, sh, cd)
    nnz = jnp.sum(adj)
    gn2 = jnp.sum(gmat * gmat)
    sfull = sh.astype(F32) + sl.astype(F32)
    ent = jnp.sum(-sfull * jnp.log(sfull + 1e-15))
    px_ref[0] = px
    padj_ref[0] = padj
    lane = lax.broadcasted_iota(jnp.int32, (1, 1, 8), 2)
    st_ref[...] = jnp.where(lane == 0, nnz, jnp.where(lane == 1, gn2, ent))


def _diffpool(A, s_hi, s_lo, x_hi, x_lo):
    return pl.pallas_call(
        _diffpool_body,
        grid=(B,),
        in_specs=[pl.BlockSpec((1, NPG, PAD), lambda i: (i, 0, 0)),
                  pl.BlockSpec((1, NPG, K), lambda i: (i, 0, 0)),
                  pl.BlockSpec((1, NPG, K), lambda i: (i, 0, 0)),
                  pl.BlockSpec((1, NPG, 256), lambda i: (i, 0, 0)),
                  pl.BlockSpec((1, NPG, 256), lambda i: (i, 0, 0))],
        out_specs=[pl.BlockSpec((1, K, 256), lambda i: (i, 0, 0)),
                   pl.BlockSpec((1, K, K), lambda i: (i, 0, 0)),
                   pl.BlockSpec((1, 1, 8), lambda i: (i, 0, 0))],
        out_shape=[jax.ShapeDtypeStruct((B, K, 256), F32),
                   jax.ShapeDtypeStruct((B, K, K), F32),
                   jax.ShapeDtypeStruct((B, 1, 8), F32)],
    )(A, s_hi.reshape(B, NPG, K), s_lo.reshape(B, NPG, K),
      x_hi.reshape(B, NPG, 256), x_lo.reshape(B, NPG, 256))


def _dh(a, b, dims):
    return lax.dot_general(a, b, (dims, ((), ())),
                           precision=lax.Precision.HIGHEST,
                           preferred_element_type=F32)


def _head_body(px_ref, padj_ref, st_ref,
               c21wl, c21wr, c21b, n21g, n21b,
               c22wl, c22wr, c22b, n22g, n22b,
               fc1w, fc1b, bn1g, bn1b,
               fc2w, fc2b, bn2g, bn2b,
               fc3w, fc3b,
               out_ref, reg_ref):
    padj = padj_ref[...]
    mask = (padj > 0).astype(F32)
    deg2 = jnp.maximum(jnp.sum(mask, axis=-1, keepdims=True), 1.0)
    px = px_ref[...]

    rdeg2 = _recip(deg2)

    def pool(mk, hflat):
        hg = hflat.reshape(B, K, 256)
        rows = [_dh(mk[g], hg[g], ((1,), (0,))) for g in range(B)]
        return jnp.concatenate(rows, axis=0).reshape(B, K, 256) * rdeg2

    mean1 = pool(mask, px).reshape(B * K, 256)
    y = (_dh(mean1, c21wl[...], ((1,), (0,)))
         + _dh(px, c21wr[...], ((1,), (0,))) + c21b[...])
    x21 = _bn(y, n21g[...], n21b[...])

    mean2 = pool(mask, x21).reshape(B * K, 256)
    y = (_dh(mean2, c22wl[...], ((1,), (0,)))
         + _dh(x21, c22wr[...], ((1,), (0,))) + c22b[...])
    x22 = _bn(y, n22g[...], n22b[...])

    co = jax.nn.relu(x22.reshape(B, K * 256))
    h = _bn(_dh(co, fc1w[...], ((1,), (0,))) + fc1b[...],
            bn1g[...], bn1b[...])
    h = _bn(_dh(jax.nn.relu(h), fc2w[...], ((1,), (0,))) + fc2b[...],
            bn2g[...], bn2b[...])
    out_ref[...] = _dh(jax.nn.relu(h), fc3w[...], ((1,), (0,))) + fc3b[...]

    st = st_ref[...]
    r4 = lax.broadcasted_iota(jnp.int32, (1, K, K), 1)
    c4 = lax.broadcasted_iota(jnp.int32, (1, K, K), 2)
    tr = jnp.sum(jnp.where(r4 == c4, padj, 0.0))
    link2 = jnp.sum(st[:, 0]) - 2.0 * tr + jnp.sum(st[:, 1])
    link = link2 * _rsqrt(link2 + 1e-30) * (1.0 / (B * NPG * NPG))
    ent = jnp.sum(st[:, 2]) * (1.0 / N)
    reg = (link * 1000.0 + ent) * 100.0
    lane = lax.broadcasted_iota(jnp.int32, (1, 8), 1)
    reg_ref[...] = jnp.where(lane == 0, reg, 0.0)


def _head(px, padj, st, p):
    out, reg = pl.pallas_call(
        _head_body,
        out_shape=[jax.ShapeDtypeStruct((B, 6), F32),
                   jax.ShapeDtypeStruct((1, 8), F32)],
    )(px.reshape(B * K, 256), padj, st.reshape(B, 8),
      p['c21_Wl'], p['c21_Wr'], p['c21_b'], p['n21_g'], p['n21_b'],
      p['c22_Wl'], p['c22_Wr'], p['c22_b'], p['n22_g'], p['n22_b'],
      p['fc1_W'], p['fc1_b'], p['bn1_g'], p['bn1_b'],
      p['fc2_W'], p['fc2_b'], p['bn2_g'], p['bn2_b'],
      p['fc3_W'], p['fc3_b'])
    return out, reg[0, 0]


def kernel(x, edge_index, params):
    p = params
    A = _build_counts(edge_index)

    xa = jnp.concatenate([x, jnp.ones((N, 1), F32)], axis=1)
    agg0 = _agg0(A, xa)
    deg = agg0[:, :, 3:]
    agg0f = agg0.reshape(N, 4)

    wl2 = jnp.concatenate([p['c11_Wl'], p['p11_Wl']], axis=1)
    wr2 = jnp.concatenate([p['c11_Wr'], p['p11_Wr']], axis=1)
    b2 = jnp.concatenate([p['c11_b'], p['p11_b']])
    g2 = jnp.concatenate([p['n11_g'], p['np11_g']])
    bb2 = jnp.concatenate([p['n11_b'], p['np11_b']])
    xs_hi, xs_lo = _layer1(agg0f, x, wl2, wr2, b2, g2, bb2)
    x11_hi, x11_lo = xs_hi[:, :256], xs_lo[:, :256]
    s11_hi, s11_lo = xs_hi[:, 256:], xs_lo[:, 256:]

    m1_hi, m1_lo = _aggmean(A, xs_hi, xs_lo, deg)
    x12_hi, x12_lo = _layer(m1_hi[:, :256], m1_lo[:, :256], x11_hi, x11_lo,
                            p['c12_Wl'], p['c12_Wr'], p['c12_b'],
                            p['n12_g'], p['n12_b'])
    s_hi, s_lo = _assign(m1_hi[:, 256:], m1_lo[:, 256:], s11_hi, s11_lo,
                         p['p12_Wl'], p['p12_Wr'], p['p12_b'],
                         p['np12_g'], p['np12_b'])

    m2_hi, m2_lo = _aggmean(A, x12_hi, x12_lo, deg)
    x13_hi, x13_lo = _layer(m2_hi, m2_lo, x12_hi, x12_lo,
                            p['c13_Wl'], p['c13_Wr'], p['c13_b'],
                            p['n13_g'], p['n13_b'])

    px, padj, st = _diffpool(A, s_hi, s_lo, x13_hi, x13_lo)
    return _head(px, padj, st, p)
